# Initial kernel scaffold; baseline (speedup 1.0000x reference)
import jax
import jax.numpy as jnp
from jax.experimental import pallas as pl


def kernel(features, W, gamma, beta):
    raise NotImplementedError("write your pallas kernel here")


# speedup vs baseline: 4.3525x; 4.3525x over previous
import functools

import jax
import jax.numpy as jnp
from jax import lax
from jax.experimental import pallas as pl
from jax.experimental.pallas import tpu as pltpu
from jax.experimental.pallas import tpu_sc as plsc

_B, _C, _N, _K, _DO = 4, 64, 2048, 20, 128
_R = 256
_NB = 128
_NBK = _NB * _K
_TOT = _B * _N * _K
_G = 128


def _sq_body(xb_ref, sq_ref):
    xb = xb_ref[0]
    xsq = xb * xb
    sq_ref[0] = jnp.sum(xsq, axis=0, keepdims=True)


def _run_sq(x):
    return pl.pallas_call(
        _sq_body,
        grid=(_B,),
        in_specs=[pl.BlockSpec((1, _C, _N), lambda b: (b, 0, 0))],
        out_specs=pl.BlockSpec((1, 1, _N), lambda b: (b, 0, 0)),
        out_shape=jax.ShapeDtypeStruct((_B, 1, _N), jnp.float32),
    )(x)


def _topk_proj_body(xt_ref, xb_ref, xxr_ref, xxc_ref, awt_ref, bwt_ref,
                    idx_ref, dist_ref, ut_ref, vt_ref, p_scr):
    b = pl.program_id(0)
    xt = xt_ref[0]
    xb = xb_ref[0]
    inner = lax.dot_general(xt, xb, (((1,), (0,)), ((), ())),
                            preferred_element_type=jnp.float32)
    p_scr[...] = (2.0 * inner - xxr_ref[0]) - xxc_ref[0]
    iota = lax.broadcasted_iota(jnp.int32, (_R, _N), 1)
    vals, inds = [], []
    for _t in range(_K):
        p = p_scr[...]
        m = jnp.max(p, axis=1, keepdims=True)
        pos = jnp.min(jnp.where(p == m, iota, _N), axis=1, keepdims=True)
        vals.append(m)
        inds.append(pos)
        p_scr[...] = jnp.where(iota == pos, -jnp.inf, p)
    dist_ref[0] = jnp.concatenate(vals, axis=1)
    idx_ref[0] = jnp.concatenate(inds, axis=1) + b * _N
    ut_ref[...] = lax.dot_general(xt, awt_ref[...], (((1,), (0,)), ((), ())),
                                  preferred_element_type=jnp.float32,
                                  precision=lax.Precision.HIGHEST)
    vt_ref[...] = lax.dot_general(xt, bwt_ref[...], (((1,), (0,)), ((), ())),
                                  preferred_element_type=jnp.float32,
                                  precision=lax.Precision.HIGHEST)


def _run_topk_proj(xT, x, xxr, xxc, awt, bwt):
    nblk = _N // _R
    return pl.pallas_call(
        _topk_proj_body,
        grid=(_B, nblk),
        in_specs=[
            pl.BlockSpec((1, _R, _C), lambda b, r: (b, r, 0)),
            pl.BlockSpec((1, _C, _N), lambda b, r: (b, 0, 0)),
            pl.BlockSpec((1, 1, _N), lambda b, r: (b, 0, 0)),
            pl.BlockSpec((1, _R, 1), lambda b, r: (b, r, 0)),
            pl.BlockSpec((_C, _DO), lambda b, r: (0, 0)),
            pl.BlockSpec((_C, _DO), lambda b, r: (0, 0)),
        ],
        out_specs=[
            pl.BlockSpec((1, _R, _K), lambda b, r: (b, r, 0)),
            pl.BlockSpec((1, _R, _K), lambda b, r: (b, r, 0)),
            pl.BlockSpec((_R, _DO), lambda b, r: (b * (_N // _R) + r, 0)),
            pl.BlockSpec((_R, _DO), lambda b, r: (b * (_N // _R) + r, 0)),
        ],
        out_shape=[
            jax.ShapeDtypeStruct((_B, _N, _K), jnp.int32),
            jax.ShapeDtypeStruct((_B, _N, _K), jnp.float32),
            jax.ShapeDtypeStruct((_B * _N, _DO), jnp.float32),
            jax.ShapeDtypeStruct((_B * _N, _DO), jnp.float32),
        ],
        scratch_shapes=[pltpu.VMEM((_R, _N), jnp.float32)],
    )(xT, x, xxr, xxc, awt, bwt)


def _sc_gather_body(nw, vt_hbm, idx_hbm, out_hbm, idx_v, rows_v, sem):
    t_per_w = _TOT // nw
    wid = lax.axis_index("s") * 2 + lax.axis_index("c")
    base = wid * t_per_w

    @pl.loop(0, t_per_w // _G)
    def _(i):
        off = base + i * _G
        pltpu.sync_copy(idx_hbm.at[pl.ds(off, _G)], idx_v)
        pltpu.async_copy(vt_hbm.at[idx_v], rows_v, sem).wait()
        pltpu.sync_copy(rows_v, out_hbm.at[pl.ds(off, _G)])


def _run_sc_gather(vt, idx_flat):
    info = plsc.get_sparse_core_info()
    nc, ns = info.num_cores, info.num_subcores
    mesh = plsc.VectorSubcoreMesh(core_axis_name="c", subcore_axis_name="s")
    body = functools.partial(_sc_gather_body, nc * ns)
    k = pl.kernel(
        body,
        out_type=jax.ShapeDtypeStruct((_TOT, _DO), jnp.float32),
        mesh=mesh,
        scratch_types=[
            pltpu.VMEM((_G,), jnp.int32),
            pltpu.VMEM((_G, _DO), jnp.float32),
            pltpu.SemaphoreType.DMA,
        ],
    )
    return k(vt, idx_flat)


def _stats_body(tmp_ref, ut_ref, d_ref, e_ref, s4_ref, out_ref):
    i = pl.program_id(0)
    u_exp = lax.dot_general(e_ref[...], ut_ref[...], (((1,), (0,)), ((), ())),
                            preferred_element_type=jnp.float32,
                            precision=lax.Precision.HIGHEST)
    d16 = d_ref[...].astype(jnp.bfloat16).astype(jnp.float32)
    y = tmp_ref[...] + u_exp + d16 * s4_ref[...]
    blk = jnp.concatenate([jnp.sum(y, axis=0, keepdims=True),
                           jnp.sum(y * y, axis=0, keepdims=True)], axis=0)

    @pl.when(i == 0)
    def _():
        out_ref[...] = blk

    @pl.when(i > 0)
    def _():
        out_ref[...] += blk


def _run_stats(tmp, ut, dcol, e, s4r):
    nblk = _TOT // _NBK
    return pl.pallas_call(
        _stats_body,
        grid=(nblk,),
        in_specs=[
            pl.BlockSpec((_NBK, _DO), lambda i: (i, 0)),
            pl.BlockSpec((_NB, _DO), lambda i: (i, 0)),
            pl.BlockSpec((_NBK, 1), lambda i: (i, 0)),
            pl.BlockSpec((_NBK, _NB), lambda i: (0, 0)),
            pl.BlockSpec((1, _DO), lambda i: (0, 0)),
        ],
        out_specs=pl.BlockSpec((2, _DO), lambda i: (0, 0)),
        out_shape=jax.ShapeDtypeStruct((2, _DO), jnp.float32),
    )(tmp, ut, dcol, e, s4r)


def _emit_body(tmp_ref, ut_ref, d_ref, e_ref, s4_ref, st_ref, g_ref, be_ref,
               out_ref):
    st = st_ref[...]
    mean = st[0:1, :] * (1.0 / _TOT)
    var = st[1:2, :] * (1.0 / _TOT) - mean * mean
    scale = g_ref[...] * lax.rsqrt(var + 1e-5)
    bias = be_ref[...] - mean * scale
    u_exp = lax.dot_general(e_ref[...], ut_ref[...], (((1,), (0,)), ((), ())),
                            preferred_element_type=jnp.float32,
                            precision=lax.Precision.HIGHEST)
    d16 = d_ref[...].astype(jnp.bfloat16).astype(jnp.float32)
    y = tmp_ref[...] + u_exp + d16 * s4_ref[...]
    z = y * scale + bias
    z = jnp.where(z > 0, z, 0.2 * z)
    out_ref[0] = z.T


def _run_emit(tmp, ut, dcol, e, s4r, stats, gr, br):
    nblk = _N // _NB
    return pl.pallas_call(
        _emit_body,
        grid=(_B, nblk),
        in_specs=[
            pl.BlockSpec((_NBK, _DO), lambda b, r: (b * (_N // _NB) + r, 0)),
            pl.BlockSpec((_NB, _DO), lambda b, r: (b * (_N // _NB) + r, 0)),
            pl.BlockSpec((_NBK, 1), lambda b, r: (b * (_N // _NB) + r, 0)),
            pl.BlockSpec((_NBK, _NB), lambda b, r: (0, 0)),
            pl.BlockSpec((1, _DO), lambda b, r: (0, 0)),
            pl.BlockSpec((2, _DO), lambda b, r: (0, 0)),
            pl.BlockSpec((1, _DO), lambda b, r: (0, 0)),
            pl.BlockSpec((1, _DO), lambda b, r: (0, 0)),
        ],
        out_specs=pl.BlockSpec((1, _DO, _NBK), lambda b, r: (b, 0, r)),
        out_shape=jax.ShapeDtypeStruct((_B, _DO, _N * _K), jnp.float32),
    )(tmp, ut, dcol, e, s4r, stats, gr, br)


def kernel(features, W, gamma, beta):
    x = jnp.squeeze(features, -1)
    xT = jnp.swapaxes(x, 1, 2)
    w = W.reshape(_DO, 4 * _C)
    awt = (w[:, :_C] - w[:, 2 * _C:3 * _C]).T
    bwt = (w[:, _C:2 * _C] + w[:, 2 * _C:3 * _C]).T
    w4bits = lax.bitcast_convert_type(w[:, 3 * _C:], jnp.int32)
    w4r = w4bits + jnp.int32(0x7FFF) + jnp.bitwise_and(
        lax.shift_right_logical(w4bits, 16), jnp.int32(1))
    w4b = lax.bitcast_convert_type(
        jnp.bitwise_and(w4r, jnp.int32(-65536)), jnp.float32)
    s4r = jnp.sum(w4b, axis=1).reshape(1, _DO)

    sq = _run_sq(x)
    idx, dist, ut, vt = _run_topk_proj(
        xT, x, sq.reshape(_B, 1, _N), sq.reshape(_B, _N, 1), awt, bwt)
    tmp = _run_sc_gather(vt, idx.reshape(_TOT))

    dcol = dist.reshape(_TOT, 1)
    rows = lax.broadcasted_iota(jnp.int32, (_NBK, _NB), 0) // _K
    cols = lax.broadcasted_iota(jnp.int32, (_NBK, _NB), 1)
    e = (rows == cols).astype(jnp.float32)

    stats = _run_stats(tmp, ut, dcol, e, s4r)
    out3 = _run_emit(tmp, ut, dcol, e, s4r, stats,
                     gamma.reshape(1, _DO), beta.reshape(1, _DO))
    return out3.reshape(_B, _DO, _N, _K)
